# baseline (speedup 1.0000x reference)
import jax
import jax.numpy as jnp
from jax.experimental import pallas as pl

_B = 8
_V = 4096
_L = _V + 2
_H = 256
_SENT = 8192
_K = 32
_NT = 32


def _loss_kernel(tf_ref, invp_ref, emb_ref, wiht_ref, whh_ref, bih_ref,
                 bhh_ref, wout_ref, woutb_ref, boutc_ref, bout_ref, out_ref):
    tf = tf_ref[:, :]
    invp = invp_ref[:, :]
    ps = jnp.where(tf != 0.0, jnp.broadcast_to(invp, (_B, _V)),
                   jnp.int32(_SENT))
    bih = bih_ref[:, :]
    bhh = bhh_ref[:, :]
    whh = whh_ref[:, :]

    def cond(carry):
        p, _, _, _ = carry
        return jnp.any(p < _L - 1)

    def body(carry):
        p, h, acc, cnt = carry

        nxt_list, pin_list = [], [p]
        for _ in range(_K):
            cand = jnp.where(ps > p, ps, jnp.int32(_SENT))
            nxt = jnp.min(cand, axis=1, keepdims=True)
            last = jnp.where(p < _L - 1, jnp.int32(_L - 1), jnp.int32(_SENT))
            nxt = jnp.minimum(nxt, last)
            nxt_list.append(nxt)
            p = jnp.minimum(nxt, jnp.int32(_L - 1))
            pin_list.append(p)
        pscal = [[q[b, 0] for b in range(_B)] for q in pin_list]

        hs_list, tl_list = [], []
        for k in range(_K):
            rows = [wiht_ref[pl.ds(pscal[k][b], 1), :] for b in range(_B)]
            gi = jnp.concatenate(rows, axis=0) + bih
            gh = jax.lax.dot_general(h, whh, (((1,), (1,)), ((), ())),
                                     preferred_element_type=jnp.float32) + bhh
            r = jax.nn.sigmoid(gi[:, :_H] + gh[:, :_H])
            z = jax.nn.sigmoid(gi[:, _H:2 * _H] + gh[:, _H:2 * _H])
            n = jnp.tanh(gi[:, 2 * _H:] + r * gh[:, 2 * _H:])
            h = (1.0 - z) * n + z * h
            hb = h.astype(jnp.bfloat16)

            trows = jnp.concatenate(
                [wout_ref[pl.ds(pscal[k + 1][b], 1), :] for b in range(_B)],
                axis=0).astype(jnp.bfloat16)
            tb = jnp.concatenate(
                [boutc_ref[pl.ds(pscal[k + 1][b], 1), :] for b in range(_B)],
                axis=0)
            tl = (jnp.sum((hb * trows).astype(jnp.float32), axis=1,
                          keepdims=True) + tb)

            hs_list.append(hb)
            tl_list.append(tl)

        hs = jnp.concatenate(hs_list, axis=0)
        nxts = jnp.concatenate(nxt_list, axis=0)
        tls = jnp.concatenate(tl_list, axis=0)
        active = nxts < _SENT

        mrun = jnp.full((_K * _B, 1), -1e30, jnp.float32)
        srun = jnp.zeros((_K * _B, 1), jnp.float32)
        for c in range(_NT + 1):
            w = 128 if c < _NT else _L - _NT * 128
            wt = woutb_ref[pl.ds(c * 128, w), :]
            part = jax.lax.dot_general(hs, wt, (((1,), (1,)), ((), ())),
                                       preferred_element_type=jnp.float32)
            part = part + bout_ref[:, c * 128:c * 128 + w]
            mnew = jnp.maximum(mrun, jnp.max(part, axis=1, keepdims=True))
            srun = (srun * jnp.exp(mrun - mnew)
                    + jnp.sum(jnp.exp(part - mnew), axis=1, keepdims=True))
            mrun = mnew
        lse = mrun + jnp.log(srun)

        contrib = jnp.where(active, lse - tls, 0.0)
        acc = acc + jnp.sum(contrib)
        cnt = cnt + jnp.sum(active.astype(jnp.float32))
        return (p, h, acc, cnt)

    p0 = jnp.zeros((_B, 1), jnp.int32)
    h0 = emb_ref[:, :]
    acc0 = jnp.zeros((1, 1), jnp.float32)
    cnt0 = jnp.zeros((1, 1), jnp.float32)
    _, _, acc, cnt = jax.lax.while_loop(cond, body, (p0, h0, acc0, cnt0))
    out_ref[:, :] = acc / cnt


def kernel(embeds, true_fp, cand_fp, batch_ptr, labels, labelfreq_sort,
           W_ih, W_hh, b_ih, b_hh, W_out, b_out):
    del cand_fp, batch_ptr, labels
    invp1 = (jnp.zeros((_V,), jnp.int32)
             .at[labelfreq_sort].set(jnp.arange(1, _V + 1, dtype=jnp.int32))
             .reshape(1, _V))
    wiht = W_ih.T
    woutb = W_out.astype(jnp.bfloat16)
    boutc = b_out.reshape(_L, 1)
    bout = b_out.reshape(1, _L)
    bih = b_ih.reshape(1, 3 * _H)
    bhh = b_hh.reshape(1, 3 * _H)

    out = pl.pallas_call(
        _loss_kernel,
        out_shape=jax.ShapeDtypeStruct((1, 1), jnp.float32),
    )(true_fp, invp1, embeds, wiht, W_hh, bih, bhh, W_out, woutb, boutc,
      bout)
    return out[0, 0]


# speedup vs baseline: 1.0399x; 1.0399x over previous
import jax
import jax.numpy as jnp
from jax.experimental import pallas as pl

_B = 8
_V = 4096
_L = _V + 2
_H = 256
_SENT = 8192
_K = 32
_NT = 32


def _loss_kernel(tf_ref, invp_ref, emb_ref, wiht_ref, whh_ref, bih_ref,
                 bhh_ref, wout_ref, boutc_ref, bout_ref, out_ref):
    tf = tf_ref[:, :]
    invp = invp_ref[:, :]
    ps = jnp.where(tf != 0.0, jnp.broadcast_to(invp, (_B, _V)),
                   jnp.int32(_SENT))
    bih = bih_ref[:, :]
    bhh = bhh_ref[:, :]
    whh = whh_ref[:, :]

    def cond(carry):
        p, _, _, _ = carry
        return jnp.any(p < _L - 1)

    def body(carry):
        p, h, acc, cnt = carry
        pscal = [p[b, 0] for b in range(_B)]
        hs_list, nxt_list, tl_list = [], [], []
        for _ in range(_K):
            cand = jnp.where(ps > p, ps, jnp.int32(_SENT))
            nxt = jnp.min(cand, axis=1, keepdims=True)
            last = jnp.where(p < _L - 1, jnp.int32(_L - 1), jnp.int32(_SENT))
            nxt = jnp.minimum(nxt, last)

            rows = [wiht_ref[pl.ds(pscal[b], 1), :] for b in range(_B)]
            gi = jnp.concatenate(rows, axis=0) + bih
            gh = jax.lax.dot_general(h, whh, (((1,), (1,)), ((), ())),
                                     preferred_element_type=jnp.float32) + bhh
            r = jax.nn.sigmoid(gi[:, :_H] + gh[:, :_H])
            z = jax.nn.sigmoid(gi[:, _H:2 * _H] + gh[:, _H:2 * _H])
            n = jnp.tanh(gi[:, 2 * _H:] + r * gh[:, 2 * _H:])
            h = (1.0 - z) * n + z * h

            p = jnp.minimum(nxt, jnp.int32(_L - 1))
            pscal = [p[b, 0] for b in range(_B)]
            trows = jnp.concatenate(
                [wout_ref[pl.ds(pscal[b], 1), :] for b in range(_B)], axis=0)
            tb = jnp.concatenate(
                [boutc_ref[pl.ds(pscal[b], 1), :] for b in range(_B)], axis=0)
            tl = jnp.sum(h * trows, axis=1, keepdims=True) + tb

            hs_list.append(h)
            nxt_list.append(nxt)
            tl_list.append(tl)

        hs = jnp.concatenate(hs_list, axis=0)
        nxts = jnp.concatenate(nxt_list, axis=0)
        tls = jnp.concatenate(tl_list, axis=0)
        active = nxts < _SENT

        mrun = jnp.full((_K * _B, 1), -1e30, jnp.float32)
        srun = jnp.zeros((_K * _B, 1), jnp.float32)
        for c in range(_NT + 1):
            w = 128 if c < _NT else _L - _NT * 128
            wt = wout_ref[pl.ds(c * 128, w), :]
            part = jax.lax.dot_general(hs, wt, (((1,), (1,)), ((), ())),
                                       preferred_element_type=jnp.float32)
            part = part + bout_ref[:, c * 128:c * 128 + w]
            mnew = jnp.maximum(mrun, jnp.max(part, axis=1, keepdims=True))
            srun = (srun * jnp.exp(mrun - mnew)
                    + jnp.sum(jnp.exp(part - mnew), axis=1, keepdims=True))
            mrun = mnew
        lse = mrun + jnp.log(srun)

        contrib = jnp.where(active, lse - tls, 0.0)
        acc = acc + jnp.sum(contrib)
        cnt = cnt + jnp.sum(active.astype(jnp.float32))
        return (p, h, acc, cnt)

    p0 = jnp.zeros((_B, 1), jnp.int32)
    h0 = emb_ref[:, :]
    acc0 = jnp.zeros((1, 1), jnp.float32)
    cnt0 = jnp.zeros((1, 1), jnp.float32)
    _, _, acc, cnt = jax.lax.while_loop(cond, body, (p0, h0, acc0, cnt0))
    out_ref[:, :] = acc / cnt


def kernel(embeds, true_fp, cand_fp, batch_ptr, labels, labelfreq_sort,
           W_ih, W_hh, b_ih, b_hh, W_out, b_out):
    del cand_fp, batch_ptr, labels
    invp1 = (jnp.zeros((_V,), jnp.int32)
             .at[labelfreq_sort].set(jnp.arange(1, _V + 1, dtype=jnp.int32))
             .reshape(1, _V))
    wiht = W_ih.T
    boutc = b_out.reshape(_L, 1)
    bout = b_out.reshape(1, _L)
    bih = b_ih.reshape(1, 3 * _H)
    bhh = b_hh.reshape(1, 3 * _H)

    out = pl.pallas_call(
        _loss_kernel,
        out_shape=jax.ShapeDtypeStruct((1, 1), jnp.float32),
    )(true_fp, invp1, embeds, wiht, W_hh, bih, bhh, W_out, boutc, bout)
    return out[0, 0]


# speedup vs baseline: 1.1726x; 1.1276x over previous
import jax
import jax.numpy as jnp
from jax.experimental import pallas as pl

_B = 8
_V = 4096
_L = _V + 2
_H = 256
_SENT = 8192
_K = 32
_NT = 32


def _loss_kernel(tf_ref, invp_ref, emb_ref, wiht_ref, whh_ref, bih_ref,
                 bhh_ref, wout_ref, boutc_ref, bout_ref, out_ref):
    tf = tf_ref[:, :]
    invp = invp_ref[:, :]
    ps = jnp.where(tf != 0.0, jnp.broadcast_to(invp, (_B, _V)),
                   jnp.int32(_SENT))
    bih = bih_ref[:, :]
    bhh = bhh_ref[:, :]
    whh = whh_ref[:, :]

    def cond(carry):
        p, _, _, _ = carry
        return jnp.any(p < _L - 1)

    def body(carry):
        p, h, acc, cnt = carry
        pscal = [p[b, 0] for b in range(_B)]
        hs_list, nxt_list, tl_list = [], [], []
        for _ in range(_K):
            cand = jnp.where(ps > p, ps, jnp.int32(_SENT))
            nxt = jnp.min(cand, axis=1, keepdims=True)
            last = jnp.where(p < _L - 1, jnp.int32(_L - 1), jnp.int32(_SENT))
            nxt = jnp.minimum(nxt, last)

            rows = [wiht_ref[pl.ds(pscal[b], 1), :] for b in range(_B)]
            gi = jnp.concatenate(rows, axis=0) + bih
            gh = jax.lax.dot_general(h, whh, (((1,), (1,)), ((), ())),
                                     preferred_element_type=jnp.float32) + bhh
            r = jax.nn.sigmoid(gi[:, :_H] + gh[:, :_H])
            z = jax.nn.sigmoid(gi[:, _H:2 * _H] + gh[:, _H:2 * _H])
            n = jnp.tanh(gi[:, 2 * _H:] + r * gh[:, 2 * _H:])
            h = (1.0 - z) * n + z * h

            p = jnp.minimum(nxt, jnp.int32(_L - 1))
            pscal = [p[b, 0] for b in range(_B)]
            trows = jnp.concatenate(
                [wout_ref[pl.ds(pscal[b], 1), :] for b in range(_B)], axis=0)
            tb = jnp.concatenate(
                [boutc_ref[pl.ds(pscal[b], 1), :] for b in range(_B)], axis=0)
            tl = jnp.sum(h * trows, axis=1, keepdims=True) + tb

            hs_list.append(h)
            nxt_list.append(nxt)
            tl_list.append(tl)

        hs = jnp.concatenate(hs_list, axis=0)
        nxts = jnp.concatenate(nxt_list, axis=0)
        tls = jnp.concatenate(tl_list, axis=0)
        active = nxts < _SENT

        mrun = jnp.full((_K * _B, 1), -1e30, jnp.float32)
        srun = jnp.zeros((_K * _B, 1), jnp.float32)
        for c in range(_NT + 1):
            w = 128 if c < _NT else _L - _NT * 128
            wt = wout_ref[pl.ds(c * 128, w), :]
            part = jax.lax.dot_general(hs, wt, (((1,), (1,)), ((), ())),
                                       preferred_element_type=jnp.float32)
            part = part + bout_ref[:, c * 128:c * 128 + w]
            mnew = jnp.maximum(mrun, jnp.max(part, axis=1, keepdims=True))
            srun = (srun * jnp.exp(mrun - mnew)
                    + jnp.sum(jnp.exp(part - mnew), axis=1, keepdims=True))
            mrun = mnew
        lse = mrun + jnp.log(srun)

        contrib = jnp.where(active, lse - tls, 0.0)
        acc = acc + jnp.sum(contrib)
        cnt = cnt + jnp.sum(active.astype(jnp.float32))
        return (p, h, acc, cnt)

    p0 = jnp.zeros((_B, 1), jnp.int32)
    h0 = emb_ref[:, :]
    acc0 = jnp.zeros((1, 1), jnp.float32)
    cnt0 = jnp.zeros((1, 1), jnp.float32)
    _, _, acc, cnt = jax.lax.while_loop(cond, body, (p0, h0, acc0, cnt0))
    out_ref[:, :] = acc / cnt


def kernel(embeds, true_fp, cand_fp, batch_ptr, labels, labelfreq_sort,
           W_ih, W_hh, b_ih, b_hh, W_out, b_out):
    del cand_fp, batch_ptr, labels
    invp1 = (jnp.argsort(labelfreq_sort) + 1).astype(jnp.int32).reshape(1, _V)
    wiht = W_ih.T
    boutc = b_out.reshape(_L, 1)
    bout = b_out.reshape(1, _L)
    bih = b_ih.reshape(1, 3 * _H)
    bhh = b_hh.reshape(1, 3 * _H)

    out = pl.pallas_call(
        _loss_kernel,
        out_shape=jax.ShapeDtypeStruct((1, 1), jnp.float32),
    )(true_fp, invp1, embeds, wiht, W_hh, bih, bhh, W_out, boutc, bout)
    return out[0, 0]
